# baseline (speedup 1.0000x reference)
import functools

import jax
import jax.numpy as jnp
import numpy as np
from jax import lax
from jax.experimental import pallas as pl
from jax.experimental.pallas import tpu as pltpu
from jax.experimental.pallas import tpu_sc as plsc

_info = plsc.get_sparse_core_info()
_NC, _NS, _L = _info.num_cores, _info.num_subcores, _info.num_lanes
_NW = _NC * _NS


def _positional_table(seq_length, d_model):
    pos = np.arange(seq_length, dtype=np.float32)[:, None]
    two_i = np.arange(0, d_model, 2, dtype=np.float32)
    div = np.power(10000.0, two_i / d_model, dtype=np.float32)
    pe = np.zeros((seq_length, d_model), dtype=np.float32)
    pe[:, 0::2] = np.sin(pos / div)
    pe[:, 1::2] = np.cos(pos / div)
    return pe


def _packed_positional_table(seq_length, d_model):
    import ml_dtypes
    pe = _positional_table(seq_length, d_model)
    g = d_model // 32
    bits = pe.astype(ml_dtypes.bfloat16).view(np.uint16).astype(np.uint32)
    pairs = bits.reshape(seq_length, g, 2, 16)
    words = pairs[:, :, 0, :] | (pairs[:, :, 1, :] << 16)
    return jnp.asarray(
        words.reshape(seq_length * d_model // 2).view(np.int32))


@functools.partial(jax.jit, static_argnums=(2, 3, 4))
def _embed(x2, table, batch, seq, d):
    pe = _packed_positional_table(seq, d)
    s_per_w = seq // _NW
    cpb = 4
    half = s_per_w // cpb
    nsteps = batch * cpb
    mesh = plsc.VectorSubcoreMesh(core_axis_name="c", subcore_axis_name="s")

    nbuf = 6

    @functools.partial(
        pl.kernel,
        mesh=mesh,
        out_type=jax.ShapeDtypeStruct((batch * seq, d), jnp.float32),
        scratch_types=[
            pltpu.VMEM((batch * s_per_w,), jnp.int32),
            pltpu.VMEM((s_per_w * d // 2,), jnp.int32),
            pltpu.VMEM((nbuf, half, d), jnp.float32),
            pltpu.SemaphoreType.DMA((nbuf,)),
            pltpu.SemaphoreType.DMA((nbuf,)),
            pltpu.SemaphoreType.DMA,
            pltpu.SemaphoreType.DMA,
        ],
    )
    def k(x_hbm, table_hbm, pe_hbm, out_hbm,
          idx_v, pe_v, tokD, sg, sw, spe, sidx):
        wid = lax.axis_index("s") * _NC + lax.axis_index("c")
        s_base = wid * s_per_w
        cols = d // _L
        hi_mask = jnp.int32(-65536)

        cp_pe = pltpu.async_copy(
            pe_hbm.at[pl.ds(pl.multiple_of(s_base * (d // 2), 8),
                            s_per_w * d // 2)], pe_v, spe)
        idx_cps = [
            pltpu.async_copy(x_hbm.at[b, pl.ds(s_base, s_per_w)],
                             idx_v.at[pl.ds(b * s_per_w, s_per_w)], sidx)
            for b in range(batch)
        ]
        for cp in idx_cps:
            cp.wait()

        def gather(step):
            p = lax.rem(step, nbuf)
            off = pl.multiple_of(step * half, 8)
            pltpu.async_copy(
                table_hbm.at[idx_v.at[pl.ds(off, half)]],
                tokD.at[p], sg.at[p])

        def wait_gather(p):
            pltpu.make_async_copy(
                table_hbm.at[idx_v.at[pl.ds(0, half)]],
                tokD.at[p], sg.at[p]).wait()

        def wait_write(p):
            pltpu.make_async_copy(
                tokD.at[p], out_hbm.at[pl.ds(0, half)], sw.at[p]).wait()

        gather(0)
        gather(1)
        cp_pe.wait()

        def body(step, c):
            p = lax.rem(step, nbuf)
            h = lax.rem(step, cpb)

            @pl.when(step + 2 < nsteps)
            def prefetch():
                @pl.when(step >= nbuf - 2)
                def drain():
                    wait_write(lax.rem(step + 2, nbuf))
                gather(step + 2)

            wait_gather(p)
            tv = tokD.at[p]
            poff = h * half

            @plsc.parallel_loop(0, half, unroll=2)
            def add_row(r):
                row = pl.multiple_of((poff + r) * (d // 2), _L)
                for g in range(cols // 2):
                    w = pe_v[pl.ds(row + g * _L, _L)]
                    a = jax.lax.bitcast_convert_type(w << 16, jnp.float32)
                    bb = jax.lax.bitcast_convert_type(w & hi_mask,
                                                      jnp.float32)
                    plsc.addupdate(tv.at[r, pl.ds(g * 2 * _L, _L)], a)
                    plsc.addupdate(tv.at[r, pl.ds(g * 2 * _L + _L, _L)], bb)

            flat = pl.multiple_of(lax.div(step, cpb) * seq + s_base + poff, 8)
            pltpu.async_copy(tv, out_hbm.at[pl.ds(flat, half)], sw.at[p])
            return c

        lax.fori_loop(0, nsteps, body, 0)
        for t in range(nsteps - nbuf, nsteps):
            wait_write(t % nbuf)

    return k(x2, table, pe)


def kernel(x, token_table):
    batch, seq = x.shape
    vocab, d = token_table.shape
    x2 = x.astype(jnp.int32)
    out = _embed(x2, token_table, batch, seq, d)
    return out.reshape(batch, seq, d)


# speedup vs baseline: 1.0007x; 1.0007x over previous
import functools

import jax
import jax.numpy as jnp
import numpy as np
from jax import lax
from jax.experimental import pallas as pl
from jax.experimental.pallas import tpu as pltpu
from jax.experimental.pallas import tpu_sc as plsc

_info = plsc.get_sparse_core_info()
_NC, _NS, _L = _info.num_cores, _info.num_subcores, _info.num_lanes
_NW = _NC * _NS


def _positional_table(seq_length, d_model):
    pos = np.arange(seq_length, dtype=np.float32)[:, None]
    two_i = np.arange(0, d_model, 2, dtype=np.float32)
    div = np.power(10000.0, two_i / d_model, dtype=np.float32)
    pe = np.zeros((seq_length, d_model), dtype=np.float32)
    pe[:, 0::2] = np.sin(pos / div)
    pe[:, 1::2] = np.cos(pos / div)
    return pe


def _packed_positional_table(seq_length, d_model):
    import ml_dtypes
    pe = _positional_table(seq_length, d_model)
    g = d_model // 32
    bits = pe.astype(ml_dtypes.bfloat16).view(np.uint16).astype(np.uint32)
    pairs = bits.reshape(seq_length, g, 2, 16)
    words = pairs[:, :, 0, :] | (pairs[:, :, 1, :] << 16)
    return jnp.asarray(
        words.reshape(seq_length * d_model // 2).view(np.int32))


@functools.partial(jax.jit, static_argnums=(2, 3, 4))
def _embed(x2, table, batch, seq, d):
    pe = _packed_positional_table(seq, d)
    s_per_w = seq // _NW
    cpb = 4
    half = s_per_w // cpb
    nsteps = batch * cpb
    mesh = plsc.VectorSubcoreMesh(core_axis_name="c", subcore_axis_name="s")

    nbuf = 4

    @functools.partial(
        pl.kernel,
        mesh=mesh,
        out_type=jax.ShapeDtypeStruct((batch * seq, d), jnp.float32),
        scratch_types=[
            pltpu.VMEM((batch * s_per_w,), jnp.int32),
            pltpu.VMEM((s_per_w * d // 2,), jnp.int32),
            pltpu.VMEM((nbuf, half, d), jnp.float32),
            pltpu.SemaphoreType.DMA((nbuf,)),
            pltpu.SemaphoreType.DMA((nbuf,)),
            pltpu.SemaphoreType.DMA,
            pltpu.SemaphoreType.DMA,
        ],
    )
    def k(x_hbm, table_hbm, pe_hbm, out_hbm,
          idx_v, pe_v, tokD, sg, sw, spe, sidx):
        wid = lax.axis_index("s") * _NC + lax.axis_index("c")
        s_base = wid * s_per_w
        cols = d // _L
        hi_mask = jnp.int32(-65536)

        cp_pe = pltpu.async_copy(
            pe_hbm.at[pl.ds(pl.multiple_of(s_base * (d // 2), 8),
                            s_per_w * d // 2)], pe_v, spe)
        idx_cps = [
            pltpu.async_copy(x_hbm.at[b, pl.ds(s_base, s_per_w)],
                             idx_v.at[pl.ds(b * s_per_w, s_per_w)], sidx)
            for b in range(batch)
        ]
        for cp in idx_cps:
            cp.wait()

        def gather(step):
            p = lax.rem(step, nbuf)
            off = pl.multiple_of(step * half, 8)
            pltpu.async_copy(
                table_hbm.at[idx_v.at[pl.ds(off, half)]],
                tokD.at[p], sg.at[p])

        def wait_gather(p):
            pltpu.make_async_copy(
                table_hbm.at[idx_v.at[pl.ds(0, half)]],
                tokD.at[p], sg.at[p]).wait()

        def wait_write(p):
            pltpu.make_async_copy(
                tokD.at[p], out_hbm.at[pl.ds(0, half)], sw.at[p]).wait()

        gather(0)
        gather(1)
        cp_pe.wait()

        def body(step, c):
            p = lax.rem(step, nbuf)
            h = lax.rem(step, cpb)

            @pl.when(step + 2 < nsteps)
            def prefetch():
                @pl.when(step >= nbuf - 2)
                def drain():
                    wait_write(lax.rem(step + 2, nbuf))
                gather(step + 2)

            wait_gather(p)
            tv = tokD.at[p]
            poff = h * half

            @plsc.parallel_loop(0, half, unroll=2)
            def add_row(r):
                row = pl.multiple_of((poff + r) * (d // 2), _L)
                for g in range(cols // 2):
                    w = pe_v[pl.ds(row + g * _L, _L)]
                    a = jax.lax.bitcast_convert_type(w << 16, jnp.float32)
                    bb = jax.lax.bitcast_convert_type(w & hi_mask,
                                                      jnp.float32)
                    plsc.addupdate(tv.at[r, pl.ds(g * 2 * _L, _L)], a)
                    plsc.addupdate(tv.at[r, pl.ds(g * 2 * _L + _L, _L)], bb)

            flat = pl.multiple_of(lax.div(step, cpb) * seq + s_base + poff, 8)
            pltpu.async_copy(tv, out_hbm.at[pl.ds(flat, half)], sw.at[p])
            return c

        lax.fori_loop(0, nsteps, body, 0)
        for t in range(nsteps - nbuf, nsteps):
            wait_write(t % nbuf)

    return k(x2, table, pe)


def kernel(x, token_table):
    batch, seq = x.shape
    vocab, d = token_table.shape
    x2 = x.astype(jnp.int32)
    out = _embed(x2, token_table, batch, seq, d)
    return out.reshape(batch, seq, d)
